# baseline (speedup 1.0000x reference)
import functools

import jax
import jax.numpy as jnp
from jax.experimental import pallas as pl
from jax.experimental.pallas import tpu as pltpu

E = 8
TOP_K = 2
F32 = jnp.float32
BF16 = jnp.bfloat16


def _router_body(x_ref, wr_ref, logits_ref, e2_ref, r2_ref, w2_ref,
                 counts_ref, wdense_ref, carry_ref, *, nblk, tb):
    g = pl.program_id(0)
    x = x_ref[...]
    logits = jax.lax.dot_general(
        x, wr_ref[...], (((1,), (1,)), ((), ())),
        preferred_element_type=F32)
    logits_ref[...] = logits

    m = jnp.max(logits, axis=1, keepdims=True)
    p = jnp.exp(logits - m)
    rw = p / jnp.sum(p, axis=1, keepdims=True)

    iota8 = jax.lax.broadcasted_iota(jnp.int32, (tb, E), 1)
    m1 = jnp.max(rw, axis=1, keepdims=True)
    e0 = jnp.min(jnp.where(rw >= m1, iota8, E), axis=1, keepdims=True)
    sel0 = iota8 == e0
    rw2 = jnp.where(sel0, -jnp.inf, rw)
    m2 = jnp.max(rw2, axis=1, keepdims=True)
    e1 = jnp.min(jnp.where(rw2 >= m2, iota8, E), axis=1, keepdims=True)
    sel1 = iota8 == e1
    s = m1 + m2
    w0 = m1 / s
    w1 = m2 / s
    wdense_ref[...] = jnp.where(sel0, w0, 0.0) + jnp.where(sel1, w1, 0.0)

    match = (sel0 | sel1).astype(F32)
    ri = jax.lax.broadcasted_iota(jnp.int32, (tb, tb), 0)
    ci = jax.lax.broadcasted_iota(jnp.int32, (tb, tb), 1)
    tri = (ri > ci).astype(F32)
    rank_in = jax.lax.dot_general(
        tri, match, (((1,), (0,)), ((), ())), preferred_element_type=F32)

    @pl.when(g == 0)
    def _():
        carry_ref[...] = jnp.zeros_like(carry_ref)

    carry = carry_ref[0:1, 0:E]
    rank_g = rank_in + carry
    colsum = jnp.sum(match, axis=0, keepdims=True)
    carry_ref[0:1, 0:E] = carry + colsum

    @pl.when(g == nblk - 1)
    def _():
        tot = carry + colsum
        counts_ref[...] = jnp.concatenate(
            [tot, jnp.zeros((1, 16 - E), F32)], axis=1).astype(jnp.int32)

    rank0 = jnp.sum(jnp.where(sel0, rank_g, 0.0), axis=1)
    rank1 = jnp.sum(jnp.where(sel1, rank_g, 0.0), axis=1)
    e2_ref[...] = jnp.concatenate(
        [e0.reshape(1, tb), e1.reshape(1, tb)], axis=0)
    r2_ref[...] = jnp.concatenate(
        [rank0.reshape(1, tb), rank1.reshape(1, tb)], axis=0).astype(jnp.int32)
    w2_ref[...] = jnp.concatenate(
        [w0.reshape(1, tb), w1.reshape(1, tb)], axis=0)


def _router(x, Wr):
    T, d = x.shape
    TB = 512
    nblk = T // TB
    body = functools.partial(_router_body, nblk=nblk, tb=TB)
    return pl.pallas_call(
        body,
        grid=(nblk,),
        in_specs=[
            pl.BlockSpec((TB, d), lambda g: (g, 0)),
            pl.BlockSpec((E, d), lambda g: (0, 0)),
        ],
        out_specs=[
            pl.BlockSpec((TB, E), lambda g: (g, 0)),
            pl.BlockSpec((TOP_K, TB), lambda g: (0, g)),
            pl.BlockSpec((TOP_K, TB), lambda g: (0, g)),
            pl.BlockSpec((TOP_K, TB), lambda g: (0, g)),
            pl.BlockSpec((1, 16), lambda g: (0, 0)),
            pl.BlockSpec((TB, E), lambda g: (g, 0)),
        ],
        out_shape=[
            jax.ShapeDtypeStruct((T, E), F32),
            jax.ShapeDtypeStruct((TOP_K, T), jnp.int32),
            jax.ShapeDtypeStruct((TOP_K, T), jnp.int32),
            jax.ShapeDtypeStruct((TOP_K, T), F32),
            jax.ShapeDtypeStruct((1, 16), jnp.int32),
            jax.ShapeDtypeStruct((T, E), F32),
        ],
        scratch_shapes=[pltpu.VMEM((1, 128), F32)],
        compiler_params=pltpu.CompilerParams(
            dimension_semantics=("arbitrary",)),
    )(x, Wr)


def _dense_body(x_ref, w1_ref, w3_ref, w2_ref, wd_ref, out_ref, acc_ref,
                *, n_e, n_fb, tb):
    i_e = pl.program_id(1)
    i_fb = pl.program_id(2)

    @pl.when((i_e == 0) & (i_fb == 0))
    def _():
        acc_ref[...] = jnp.zeros_like(acc_ref)

    x = x_ref[...]
    a = jax.lax.dot_general(x, w1_ref[0], (((1,), (1,)), ((), ())),
                            preferred_element_type=F32)
    b = jax.lax.dot_general(x, w3_ref[0], (((1,), (1,)), ((), ())),
                            preferred_element_type=F32)
    h = (a * (1.0 / (1.0 + jnp.exp(-a))) * b).astype(BF16)
    part = jax.lax.dot_general(h, w2_ref[0], (((1,), (1,)), ((), ())),
                               preferred_element_type=F32)
    iota8 = jax.lax.broadcasted_iota(jnp.int32, (tb, E), 1)
    wcol = jnp.sum(jnp.where(iota8 == i_e, wd_ref[...], 0.0), axis=1,
                   keepdims=True)
    acc_ref[...] += part * wcol

    @pl.when((i_e == n_e - 1) & (i_fb == n_fb - 1))
    def _():
        out_ref[...] = acc_ref[...]


def _dense_experts(x, W1, W3, W2, wdense):
    T, d = x.shape
    f = W1.shape[1]
    TB = 256
    FB = 2048
    n_tb, n_fb = T // TB, f // FB
    xb = x.astype(BF16)
    w1b = W1.astype(BF16)
    w3b = W3.astype(BF16)
    w2b = W2.astype(BF16)
    body = functools.partial(_dense_body, n_e=E, n_fb=n_fb, tb=TB)
    return pl.pallas_call(
        body,
        grid=(n_tb, E, n_fb),
        in_specs=[
            pl.BlockSpec((TB, d), lambda t, e, fb: (t, 0)),
            pl.BlockSpec((1, FB, d), lambda t, e, fb: (e, fb, 0)),
            pl.BlockSpec((1, FB, d), lambda t, e, fb: (e, fb, 0)),
            pl.BlockSpec((1, d, FB), lambda t, e, fb: (e, 0, fb)),
            pl.BlockSpec((TB, E), lambda t, e, fb: (t, 0)),
        ],
        out_specs=pl.BlockSpec((TB, d), lambda t, e, fb: (t, 0)),
        out_shape=jax.ShapeDtypeStruct((T, d), F32),
        scratch_shapes=[pltpu.VMEM((TB, d), F32)],
        compiler_params=pltpu.CompilerParams(
            dimension_semantics=("arbitrary", "arbitrary", "arbitrary")),
    )(xb, w1b, w3b, w2b, wdense)


def kernel(hidden_states, Wr, W1, W3, W2):
    bs, S, d = hidden_states.shape
    x = hidden_states.reshape(-1, d)
    logits, e2, r2, w2p, counts, wdense = _router(x, Wr)
    final = _dense_experts(x, W1, W3, W2, wdense)
    return final.reshape(bs, S, d), logits


# speedup vs baseline: 2.3242x; 2.3242x over previous
import functools

import jax
import jax.numpy as jnp
from jax import lax
from jax.experimental import pallas as pl
from jax.experimental.pallas import tpu as pltpu
from jax.experimental.pallas import tpu_sc as plsc

E = 8
TOP_K = 2
B = 256
LOG2B = 8
F32 = jnp.float32
I32 = jnp.int32


def _router_body(x_ref, wr_ref, logits_ref, e2_ref, r2_ref, wtok_ref,
                 counts_ref, carry_ref, *, nblk, tb):
    g = pl.program_id(0)
    x = x_ref[...]
    logits = lax.dot_general(x, wr_ref[...], (((1,), (1,)), ((), ())),
                             preferred_element_type=F32)
    logits_ref[...] = logits

    m = jnp.max(logits, axis=1, keepdims=True)
    p = jnp.exp(logits - m)
    rw = p / jnp.sum(p, axis=1, keepdims=True)

    iota8 = lax.broadcasted_iota(I32, (tb, E), 1)
    m1 = jnp.max(rw, axis=1, keepdims=True)
    e0 = jnp.min(jnp.where(rw >= m1, iota8, E), axis=1, keepdims=True)
    sel0 = iota8 == e0
    rw2 = jnp.where(sel0, -jnp.inf, rw)
    m2 = jnp.max(rw2, axis=1, keepdims=True)
    e1 = jnp.min(jnp.where(rw2 >= m2, iota8, E), axis=1, keepdims=True)
    sel1 = iota8 == e1
    s = m1 + m2
    wtok_ref[...] = jnp.concatenate([m1 / s, m2 / s], axis=1)

    match = (sel0 | sel1).astype(F32)
    ri = lax.broadcasted_iota(I32, (tb, tb), 0)
    ci = lax.broadcasted_iota(I32, (tb, tb), 1)
    tri = (ri > ci).astype(F32)
    rank_in = lax.dot_general(tri, match, (((1,), (0,)), ((), ())),
                              preferred_element_type=F32)

    @pl.when(g == 0)
    def _():
        carry_ref[...] = jnp.zeros_like(carry_ref)

    carry = carry_ref[0:1, 0:E]
    rank_g = rank_in + carry
    colsum = jnp.sum(match, axis=0, keepdims=True)
    carry_ref[0:1, 0:E] = carry + colsum

    @pl.when(g == nblk - 1)
    def _():
        tot = carry + colsum
        counts_ref[...] = jnp.concatenate(
            [tot, jnp.zeros((1, 16 - E), F32)], axis=1).astype(I32)

    rank0 = jnp.sum(jnp.where(sel0, rank_g, 0.0), axis=1)
    rank1 = jnp.sum(jnp.where(sel1, rank_g, 0.0), axis=1)
    e2_ref[...] = jnp.concatenate(
        [e0.reshape(1, tb), e1.reshape(1, tb)], axis=0)
    r2_ref[...] = jnp.concatenate(
        [rank0.reshape(1, tb), rank1.reshape(1, tb)], axis=0).astype(I32)


def _router(x, Wr):
    T, d = x.shape
    TB = 512
    nblk = T // TB
    body = functools.partial(_router_body, nblk=nblk, tb=TB)
    return pl.pallas_call(
        body,
        grid=(nblk,),
        in_specs=[
            pl.BlockSpec((TB, d), lambda g: (g, 0)),
            pl.BlockSpec((E, d), lambda g: (0, 0)),
        ],
        out_specs=[
            pl.BlockSpec((TB, E), lambda g: (g, 0)),
            pl.BlockSpec((TOP_K, TB), lambda g: (0, g)),
            pl.BlockSpec((TOP_K, TB), lambda g: (0, g)),
            pl.BlockSpec((TB, TOP_K), lambda g: (g, 0)),
            pl.BlockSpec((1, 16), lambda g: (0, 0)),
        ],
        out_shape=[
            jax.ShapeDtypeStruct((T, E), F32),
            jax.ShapeDtypeStruct((TOP_K, T), I32),
            jax.ShapeDtypeStruct((TOP_K, T), I32),
            jax.ShapeDtypeStruct((T, TOP_K), F32),
            jax.ShapeDtypeStruct((1, 16), I32),
        ],
        scratch_shapes=[pltpu.VMEM((1, 128), F32)],
        compiler_params=pltpu.CompilerParams(
            dimension_semantics=("arbitrary",)),
    )(x, Wr)


def _posmeta_body(cnt_ref, e2_ref, r2_ref, pos_ref, meta_ref, *, tb, n_meta):
    g = pl.program_id(0)
    offs = []
    ends = []
    off = 0
    for e in range(E):
        ce = cnt_ref[e]
        pce = ((ce + (B - 1)) >> LOG2B) << LOG2B
        offs.append(off)
        off = off + pce
        ends.append(off)

    eblk = e2_ref[...]
    acc = r2_ref[...]
    for e in range(E):
        acc = acc + jnp.where(eblk == e, offs[e], 0)
    pos_ref[...] = acc

    @pl.when(g == 0)
    def _():
        gv = lax.broadcasted_iota(I32, (1, n_meta), 1) * B
        accm = jnp.zeros((1, n_meta), I32)
        for e in range(E):
            accm = accm + (gv >= ends[e]).astype(I32)
        bev = jnp.minimum(accm, E - 1)
        nb = lax.shift_right_logical(ends[E - 1], LOG2B)
        lane = lax.broadcasted_iota(I32, (1, n_meta), 1)
        meta_ref[...] = jnp.where(lane == n_meta - 1, nb, bev)


def _posmeta(e2, r2, cnt16, n_meta):
    T = e2.shape[1]
    TB = 512
    grid_spec = pltpu.PrefetchScalarGridSpec(
        num_scalar_prefetch=1,
        grid=(T // TB,),
        in_specs=[
            pl.BlockSpec((TOP_K, TB), lambda g, c: (0, g)),
            pl.BlockSpec((TOP_K, TB), lambda g, c: (0, g)),
        ],
        out_specs=[
            pl.BlockSpec((TOP_K, TB), lambda g, c: (0, g)),
            pl.BlockSpec((1, n_meta), lambda g, c: (0, 0)),
        ],
    )
    return pl.pallas_call(
        functools.partial(_posmeta_body, tb=TB, n_meta=n_meta),
        grid_spec=grid_spec,
        out_shape=[
            jax.ShapeDtypeStruct((TOP_K, T), I32),
            jax.ShapeDtypeStruct((1, n_meta), I32),
        ],
        compiler_params=pltpu.CompilerParams(
            dimension_semantics=("arbitrary",)),
    )(cnt16, e2, r2)


def _dispatch(x, pos_flat, ns_tot):
    T, d = x.shape
    info = plsc.get_sparse_core_info()
    NC, NSUB = info.num_cores, info.num_subcores
    NW = NC * NSUB
    tok_w = T // NW
    n_ch = tok_w // 16
    mesh = plsc.VectorSubcoreMesh(core_axis_name="c", subcore_axis_name="s")

    @functools.partial(
        pl.kernel, mesh=mesh,
        out_type=jax.ShapeDtypeStruct((ns_tot, d), F32),
        scratch_types=[
            pltpu.VMEM((16,), I32),
            pltpu.VMEM((16,), I32),
            pltpu.VMEM((16, d), F32),
            pltpu.SemaphoreType.DMA,
            pltpu.SemaphoreType.DMA,
        ],
    )
    def disp(x_hbm, pos_hbm, xs_hbm, idx0, idx1, xv, sem0, sem1):
        wid = lax.axis_index("s") * NC + lax.axis_index("c")
        base = wid * tok_w
        for c in range(n_ch):
            pltpu.sync_copy(x_hbm.at[pl.ds(base + c * 16, 16)], xv)
            pltpu.sync_copy(pos_hbm.at[pl.ds(base + c * 16, 16)], idx0)
            pltpu.sync_copy(pos_hbm.at[pl.ds(T + base + c * 16, 16)], idx1)
            cp0 = pltpu.async_copy(xv, xs_hbm.at[idx0], sem0)
            cp1 = pltpu.async_copy(xv, xs_hbm.at[idx1], sem1)
            cp0.wait()
            cp1.wait()

    return disp(x, pos_flat)


def _k1_body(meta_ref, xs_ref, w1_ref, w3_ref, h_ref, *, n_meta):
    g = pl.program_id(1)

    @pl.when(g < meta_ref[n_meta - 1])
    def _():
        x = xs_ref[...]
        a = lax.dot_general(x, w1_ref[0], (((1,), (1,)), ((), ())),
                            preferred_element_type=F32)
        b = lax.dot_general(x, w3_ref[0], (((1,), (1,)), ((), ())),
                            preferred_element_type=F32)
        h_ref[...] = a * (1.0 / (1.0 + jnp.exp(-a))) * b


def _k1(meta, xs, W1, W3, n_meta):
    ns_tot, d = xs.shape
    f = W1.shape[1]
    FB = 2048
    n_fb = f // FB
    G = ns_tot // B
    grid_spec = pltpu.PrefetchScalarGridSpec(
        num_scalar_prefetch=1,
        grid=(n_fb, G),
        in_specs=[
            pl.BlockSpec((B, d), lambda fb, g, m: (g, 0)),
            pl.BlockSpec((1, FB, d), lambda fb, g, m: (m[g], fb, 0)),
            pl.BlockSpec((1, FB, d), lambda fb, g, m: (m[g], fb, 0)),
        ],
        out_specs=pl.BlockSpec((B, FB), lambda fb, g, m: (g, fb)),
    )
    return pl.pallas_call(
        functools.partial(_k1_body, n_meta=n_meta),
        grid_spec=grid_spec,
        out_shape=jax.ShapeDtypeStruct((ns_tot, f), F32),
        compiler_params=pltpu.CompilerParams(
            dimension_semantics=("arbitrary", "arbitrary")),
    )(meta, xs, W1, W3)


def _k2_body(meta_ref, h_ref, w2_ref, out_ref, *, n_meta):
    g = pl.program_id(0)

    @pl.when(g < meta_ref[n_meta - 1])
    def _():
        out_ref[...] = lax.dot_general(
            h_ref[...], w2_ref[0], (((1,), (1,)), ((), ())),
            preferred_element_type=F32)


def _k2(meta, h, W2, n_meta):
    ns_tot, f = h.shape
    d = W2.shape[1]
    G = ns_tot // B
    grid_spec = pltpu.PrefetchScalarGridSpec(
        num_scalar_prefetch=1,
        grid=(G,),
        in_specs=[
            pl.BlockSpec((B, f), lambda g, m: (g, 0)),
            pl.BlockSpec((1, d, f), lambda g, m: (m[g], 0, 0)),
        ],
        out_specs=pl.BlockSpec((B, d), lambda g, m: (g, 0)),
    )
    return pl.pallas_call(
        functools.partial(_k2_body, n_meta=n_meta),
        grid_spec=grid_spec,
        out_shape=jax.ShapeDtypeStruct((ns_tot, d), F32),
        compiler_params=pltpu.CompilerParams(
            dimension_semantics=("arbitrary",)),
    )(meta, h, W2)


def _undispatch(osort, pos_flat, T, d):
    info = plsc.get_sparse_core_info()
    NC, NSUB = info.num_cores, info.num_subcores
    NW = NC * NSUB
    tok_w = T // NW
    n_ch = tok_w // 16
    mesh = plsc.VectorSubcoreMesh(core_axis_name="c", subcore_axis_name="s")

    @functools.partial(
        pl.kernel, mesh=mesh,
        out_type=jax.ShapeDtypeStruct((TOP_K * T, d), F32),
        scratch_types=[
            pltpu.VMEM((16,), I32),
            pltpu.VMEM((16, d), F32),
            pltpu.SemaphoreType.DMA,
        ],
    )
    def undisp(os_hbm, pos_hbm, op_hbm, idx, buf, sem):
        wid = lax.axis_index("s") * NC + lax.axis_index("c")
        base = wid * tok_w
        for c in range(2 * n_ch):
            src = base + c * 16 if c < n_ch else T + base + (c - n_ch) * 16
            pltpu.sync_copy(pos_hbm.at[pl.ds(src, 16)], idx)
            pltpu.async_copy(os_hbm.at[idx], buf, sem).wait()
            pltpu.sync_copy(buf, op_hbm.at[pl.ds(src, 16)])

    return undisp(osort, pos_flat)


def _comb_body(op0_ref, op1_ref, w_ref, out_ref):
    w = w_ref[...]
    out_ref[...] = op0_ref[0] * w[:, 0:1] + op1_ref[0] * w[:, 1:2]


def _combine(op, wtok):
    _, T, d = op.shape
    TB = 512
    return pl.pallas_call(
        _comb_body,
        grid=(T // TB,),
        in_specs=[
            pl.BlockSpec((1, TB, d), lambda g: (0, g, 0)),
            pl.BlockSpec((1, TB, d), lambda g: (1, g, 0)),
            pl.BlockSpec((TB, TOP_K), lambda g: (g, 0)),
        ],
        out_specs=pl.BlockSpec((TB, d), lambda g: (g, 0)),
        out_shape=jax.ShapeDtypeStruct((T, d), F32),
        compiler_params=pltpu.CompilerParams(
            dimension_semantics=("arbitrary",)),
    )(op, op, wtok)


def kernel(hidden_states, Wr, W1, W3, W2):
    bs, S, d = hidden_states.shape
    T = bs * S
    ns_tot = TOP_K * T + E * B
    n_meta = 64
    x = hidden_states.reshape(-1, d)
    logits, e2, r2, wtok, counts = _router(x, Wr)
    pos, meta = _posmeta(e2, r2, counts.reshape(16), n_meta)
    pos_flat = pos.reshape(-1)
    meta_flat = meta.reshape(-1)
    xs = _dispatch(x, pos_flat, ns_tot)
    h = _k1(meta_flat, xs, W1, W3, n_meta)
    osort = _k2(meta_flat, h, W2, n_meta)
    op = _undispatch(osort, pos_flat, T, d)
    final = _combine(op.reshape(TOP_K, T, d), wtok)
    return final.reshape(bs, S, d), logits


# speedup vs baseline: 2.4019x; 1.0334x over previous
import functools

import jax
import jax.numpy as jnp
from jax import lax
from jax.experimental import pallas as pl
from jax.experimental.pallas import tpu as pltpu
from jax.experimental.pallas import tpu_sc as plsc

E = 8
TOP_K = 2
B = 256
LOG2B = 8
F32 = jnp.float32
I32 = jnp.int32


def _router_body(x_ref, wr_ref, logits_ref, e2_ref, r2_ref, wtok_ref,
                 counts_ref, carry_ref, *, nblk, tb):
    g = pl.program_id(0)
    x = x_ref[...]
    logits = lax.dot_general(x, wr_ref[...], (((1,), (1,)), ((), ())),
                             preferred_element_type=F32)
    logits_ref[...] = logits

    m = jnp.max(logits, axis=1, keepdims=True)
    p = jnp.exp(logits - m)
    rw = p / jnp.sum(p, axis=1, keepdims=True)

    iota8 = lax.broadcasted_iota(I32, (tb, E), 1)
    m1 = jnp.max(rw, axis=1, keepdims=True)
    e0 = jnp.min(jnp.where(rw >= m1, iota8, E), axis=1, keepdims=True)
    sel0 = iota8 == e0
    rw2 = jnp.where(sel0, -jnp.inf, rw)
    m2 = jnp.max(rw2, axis=1, keepdims=True)
    e1 = jnp.min(jnp.where(rw2 >= m2, iota8, E), axis=1, keepdims=True)
    sel1 = iota8 == e1
    s = m1 + m2
    wtok_ref[...] = jnp.concatenate([m1 / s, m2 / s], axis=1)

    match = (sel0 | sel1).astype(F32)
    ri = lax.broadcasted_iota(I32, (tb, tb), 0)
    ci = lax.broadcasted_iota(I32, (tb, tb), 1)
    tri = (ri > ci).astype(F32)
    rank_in = lax.dot_general(tri, match, (((1,), (0,)), ((), ())),
                              preferred_element_type=F32)

    @pl.when(g == 0)
    def _():
        carry_ref[...] = jnp.zeros_like(carry_ref)

    carry = carry_ref[0:1, 0:E]
    rank_g = rank_in + carry
    colsum = jnp.sum(match, axis=0, keepdims=True)
    carry_ref[0:1, 0:E] = carry + colsum

    @pl.when(g == nblk - 1)
    def _():
        tot = carry + colsum
        counts_ref[...] = jnp.concatenate(
            [tot, jnp.zeros((1, 16 - E), F32)], axis=1).astype(I32)

    rank0 = jnp.sum(jnp.where(sel0, rank_g, 0.0), axis=1)
    rank1 = jnp.sum(jnp.where(sel1, rank_g, 0.0), axis=1)
    e2_ref[...] = jnp.concatenate(
        [e0.reshape(1, tb), e1.reshape(1, tb)], axis=0)
    r2_ref[...] = jnp.concatenate(
        [rank0.reshape(1, tb), rank1.reshape(1, tb)], axis=0).astype(I32)


def _router(x, Wr):
    T, d = x.shape
    TB = 512
    nblk = T // TB
    body = functools.partial(_router_body, nblk=nblk, tb=TB)
    return pl.pallas_call(
        body,
        grid=(nblk,),
        in_specs=[
            pl.BlockSpec((TB, d), lambda g: (g, 0)),
            pl.BlockSpec((E, d), lambda g: (0, 0)),
        ],
        out_specs=[
            pl.BlockSpec((TB, E), lambda g: (g, 0)),
            pl.BlockSpec((TOP_K, TB), lambda g: (0, g)),
            pl.BlockSpec((TOP_K, TB), lambda g: (0, g)),
            pl.BlockSpec((TB, TOP_K), lambda g: (g, 0)),
            pl.BlockSpec((1, 16), lambda g: (0, 0)),
        ],
        out_shape=[
            jax.ShapeDtypeStruct((T, E), F32),
            jax.ShapeDtypeStruct((TOP_K, T), I32),
            jax.ShapeDtypeStruct((TOP_K, T), I32),
            jax.ShapeDtypeStruct((T, TOP_K), F32),
            jax.ShapeDtypeStruct((1, 16), I32),
        ],
        scratch_shapes=[pltpu.VMEM((1, 128), F32)],
        compiler_params=pltpu.CompilerParams(
            dimension_semantics=("arbitrary",)),
    )(x, Wr)


def _posmeta_body(cnt_ref, e2_ref, r2_ref, pos_ref, meta_ref, *, tb, n_meta):
    g = pl.program_id(0)
    offs = []
    ends = []
    off = 0
    for e in range(E):
        ce = cnt_ref[e]
        pce = ((ce + (B - 1)) >> LOG2B) << LOG2B
        offs.append(off)
        off = off + pce
        ends.append(off)

    eblk = e2_ref[...]
    acc = r2_ref[...]
    for e in range(E):
        acc = acc + jnp.where(eblk == e, offs[e], 0)
    pos_ref[...] = acc

    @pl.when(g == 0)
    def _():
        gv = lax.broadcasted_iota(I32, (1, n_meta), 1) * B
        accm = jnp.zeros((1, n_meta), I32)
        for e in range(E):
            accm = accm + (gv >= ends[e]).astype(I32)
        bev = jnp.minimum(accm, E - 1)
        nb = lax.shift_right_logical(ends[E - 1], LOG2B)
        lane = lax.broadcasted_iota(I32, (1, n_meta), 1)
        meta_ref[...] = jnp.where(lane == n_meta - 1, nb, bev)


def _posmeta(e2, r2, cnt16, n_meta):
    T = e2.shape[1]
    TB = 512
    grid_spec = pltpu.PrefetchScalarGridSpec(
        num_scalar_prefetch=1,
        grid=(T // TB,),
        in_specs=[
            pl.BlockSpec((TOP_K, TB), lambda g, c: (0, g)),
            pl.BlockSpec((TOP_K, TB), lambda g, c: (0, g)),
        ],
        out_specs=[
            pl.BlockSpec((TOP_K, TB), lambda g, c: (0, g)),
            pl.BlockSpec((1, n_meta), lambda g, c: (0, 0)),
        ],
    )
    return pl.pallas_call(
        functools.partial(_posmeta_body, tb=TB, n_meta=n_meta),
        grid_spec=grid_spec,
        out_shape=[
            jax.ShapeDtypeStruct((TOP_K, T), I32),
            jax.ShapeDtypeStruct((1, n_meta), I32),
        ],
        compiler_params=pltpu.CompilerParams(
            dimension_semantics=("arbitrary",)),
    )(cnt16, e2, r2)


def _dispatch(x, pos_flat, ns_tot):
    T, d = x.shape
    info = plsc.get_sparse_core_info()
    NC, NSUB = info.num_cores, info.num_subcores
    NW = NC * NSUB
    tok_w = T // NW
    n_ch = tok_w // 16
    mesh = plsc.VectorSubcoreMesh(core_axis_name="c", subcore_axis_name="s")

    @functools.partial(
        pl.kernel, mesh=mesh,
        out_type=jax.ShapeDtypeStruct((ns_tot, d), F32),
        scratch_types=[
            pltpu.VMEM((16,), I32),
            pltpu.VMEM((16,), I32),
            pltpu.VMEM((16, d), F32),
            pltpu.SemaphoreType.DMA,
            pltpu.SemaphoreType.DMA,
        ],
    )
    def disp(x_hbm, pos_hbm, xs_hbm, idx0, idx1, xv, sem0, sem1):
        wid = lax.axis_index("s") * NC + lax.axis_index("c")
        base = wid * tok_w
        for c in range(n_ch):
            pltpu.sync_copy(x_hbm.at[pl.ds(base + c * 16, 16)], xv)
            pltpu.sync_copy(pos_hbm.at[pl.ds(base + c * 16, 16)], idx0)
            pltpu.sync_copy(pos_hbm.at[pl.ds(T + base + c * 16, 16)], idx1)
            cp0 = pltpu.async_copy(xv, xs_hbm.at[idx0], sem0)
            cp1 = pltpu.async_copy(xv, xs_hbm.at[idx1], sem1)
            cp0.wait()
            cp1.wait()

    return disp(x, pos_flat)


def _k1_body(meta_ref, xs_ref, w1_ref, w3_ref, h_ref, *, n_meta):
    g = pl.program_id(1)

    @pl.when(g < meta_ref[n_meta - 1])
    def _():
        x = xs_ref[...]
        a = lax.dot_general(x, w1_ref[0], (((1,), (1,)), ((), ())),
                            preferred_element_type=F32)
        b = lax.dot_general(x, w3_ref[0], (((1,), (1,)), ((), ())),
                            preferred_element_type=F32)
        h_ref[...] = a * (1.0 / (1.0 + jnp.exp(-a))) * b


def _k1(meta, xs, W1, W3, n_meta):
    ns_tot, d = xs.shape
    f = W1.shape[1]
    FB = 2048
    n_fb = f // FB
    G = ns_tot // B
    grid_spec = pltpu.PrefetchScalarGridSpec(
        num_scalar_prefetch=1,
        grid=(n_fb, G),
        in_specs=[
            pl.BlockSpec(
                (B, d),
                lambda fb, g, m, n=n_meta: (jnp.minimum(g, m[n - 1] - 1), 0)),
            pl.BlockSpec((1, FB, d), lambda fb, g, m: (m[g], fb, 0)),
            pl.BlockSpec((1, FB, d), lambda fb, g, m: (m[g], fb, 0)),
        ],
        out_specs=pl.BlockSpec(
            (B, FB),
            lambda fb, g, m, n=n_meta: (jnp.minimum(g, m[n - 1] - 1), fb)),
    )
    return pl.pallas_call(
        functools.partial(_k1_body, n_meta=n_meta),
        grid_spec=grid_spec,
        out_shape=jax.ShapeDtypeStruct((ns_tot, f), F32),
        compiler_params=pltpu.CompilerParams(
            dimension_semantics=("arbitrary", "arbitrary")),
    )(meta, xs, W1, W3)


def _k2_body(meta_ref, h_ref, w2_ref, out_ref, *, n_meta):
    g = pl.program_id(0)

    @pl.when(g < meta_ref[n_meta - 1])
    def _():
        out_ref[...] = lax.dot_general(
            h_ref[...], w2_ref[0], (((1,), (1,)), ((), ())),
            preferred_element_type=F32)


def _k2(meta, h, W2, n_meta):
    ns_tot, f = h.shape
    d = W2.shape[1]
    G = ns_tot // B
    grid_spec = pltpu.PrefetchScalarGridSpec(
        num_scalar_prefetch=1,
        grid=(G,),
        in_specs=[
            pl.BlockSpec(
                (B, f),
                lambda g, m, n=n_meta: (jnp.minimum(g, m[n - 1] - 1), 0)),
            pl.BlockSpec((1, d, f), lambda g, m: (m[g], 0, 0)),
        ],
        out_specs=pl.BlockSpec(
            (B, d), lambda g, m, n=n_meta: (jnp.minimum(g, m[n - 1] - 1), 0)),
    )
    return pl.pallas_call(
        functools.partial(_k2_body, n_meta=n_meta),
        grid_spec=grid_spec,
        out_shape=jax.ShapeDtypeStruct((ns_tot, d), F32),
        compiler_params=pltpu.CompilerParams(
            dimension_semantics=("arbitrary",)),
    )(meta, h, W2)


def _undispatch(osort, pos_flat, T, d):
    info = plsc.get_sparse_core_info()
    NC, NSUB = info.num_cores, info.num_subcores
    NW = NC * NSUB
    tok_w = T // NW
    n_ch = tok_w // 16
    mesh = plsc.VectorSubcoreMesh(core_axis_name="c", subcore_axis_name="s")

    @functools.partial(
        pl.kernel, mesh=mesh,
        out_type=jax.ShapeDtypeStruct((TOP_K * T, d), F32),
        scratch_types=[
            pltpu.VMEM((16,), I32),
            pltpu.VMEM((16, d), F32),
            pltpu.SemaphoreType.DMA,
        ],
    )
    def undisp(os_hbm, pos_hbm, op_hbm, idx, buf, sem):
        wid = lax.axis_index("s") * NC + lax.axis_index("c")
        base = wid * tok_w
        for c in range(2 * n_ch):
            src = base + c * 16 if c < n_ch else T + base + (c - n_ch) * 16
            pltpu.sync_copy(pos_hbm.at[pl.ds(src, 16)], idx)
            pltpu.async_copy(os_hbm.at[idx], buf, sem).wait()
            pltpu.sync_copy(buf, op_hbm.at[pl.ds(src, 16)])

    return undisp(osort, pos_flat)


def _comb_body(op0_ref, op1_ref, w_ref, out_ref):
    w = w_ref[...]
    out_ref[...] = op0_ref[0] * w[:, 0:1] + op1_ref[0] * w[:, 1:2]


def _combine(op, wtok):
    _, T, d = op.shape
    TB = 512
    return pl.pallas_call(
        _comb_body,
        grid=(T // TB,),
        in_specs=[
            pl.BlockSpec((1, TB, d), lambda g: (0, g, 0)),
            pl.BlockSpec((1, TB, d), lambda g: (1, g, 0)),
            pl.BlockSpec((TB, TOP_K), lambda g: (g, 0)),
        ],
        out_specs=pl.BlockSpec((TB, d), lambda g: (g, 0)),
        out_shape=jax.ShapeDtypeStruct((T, d), F32),
        compiler_params=pltpu.CompilerParams(
            dimension_semantics=("arbitrary",)),
    )(op, op, wtok)


def kernel(hidden_states, Wr, W1, W3, W2):
    bs, S, d = hidden_states.shape
    T = bs * S
    ns_tot = TOP_K * T + E * B
    n_meta = 64
    x = hidden_states.reshape(-1, d)
    logits, e2, r2, wtok, counts = _router(x, Wr)
    pos, meta = _posmeta(e2, r2, counts.reshape(16), n_meta)
    pos_flat = pos.reshape(-1)
    meta_flat = meta.reshape(-1)
    xs = _dispatch(x, pos_flat, ns_tot)
    h = _k1(meta_flat, xs, W1, W3, n_meta)
    osort = _k2(meta_flat, h, W2, n_meta)
    op = _undispatch(osort, pos_flat, T, d)
    final = _combine(op.reshape(TOP_K, T, d), wtok)
    return final.reshape(bs, S, d), logits


# speedup vs baseline: 2.4631x; 1.0255x over previous
import functools

import jax
import jax.numpy as jnp
from jax import lax
from jax.experimental import pallas as pl
from jax.experimental.pallas import tpu as pltpu
from jax.experimental.pallas import tpu_sc as plsc

E = 8
TOP_K = 2
B = 512
LOG2B = 9
F32 = jnp.float32
I32 = jnp.int32


def _router_body(x_ref, wr_ref, logits_ref, e2_ref, r2_ref, wtok_ref,
                 counts_ref, carry_ref, *, nblk, tb):
    g = pl.program_id(0)
    x = x_ref[...]
    logits = lax.dot_general(x, wr_ref[...], (((1,), (1,)), ((), ())),
                             preferred_element_type=F32)
    logits_ref[...] = logits

    m = jnp.max(logits, axis=1, keepdims=True)
    p = jnp.exp(logits - m)
    rw = p / jnp.sum(p, axis=1, keepdims=True)

    iota8 = lax.broadcasted_iota(I32, (tb, E), 1)
    m1 = jnp.max(rw, axis=1, keepdims=True)
    e0 = jnp.min(jnp.where(rw >= m1, iota8, E), axis=1, keepdims=True)
    sel0 = iota8 == e0
    rw2 = jnp.where(sel0, -jnp.inf, rw)
    m2 = jnp.max(rw2, axis=1, keepdims=True)
    e1 = jnp.min(jnp.where(rw2 >= m2, iota8, E), axis=1, keepdims=True)
    sel1 = iota8 == e1
    s = m1 + m2
    wtok_ref[...] = jnp.concatenate([m1 / s, m2 / s], axis=1)

    match = (sel0 | sel1).astype(F32)
    ri = lax.broadcasted_iota(I32, (tb, tb), 0)
    ci = lax.broadcasted_iota(I32, (tb, tb), 1)
    tri = (ri > ci).astype(F32)
    rank_in = lax.dot_general(tri, match, (((1,), (0,)), ((), ())),
                              preferred_element_type=F32)

    @pl.when(g == 0)
    def _():
        carry_ref[...] = jnp.zeros_like(carry_ref)

    carry = carry_ref[0:1, 0:E]
    rank_g = rank_in + carry
    colsum = jnp.sum(match, axis=0, keepdims=True)
    carry_ref[0:1, 0:E] = carry + colsum

    @pl.when(g == nblk - 1)
    def _():
        tot = carry + colsum
        counts_ref[...] = jnp.concatenate(
            [tot, jnp.zeros((1, 16 - E), F32)], axis=1).astype(I32)

    rank0 = jnp.sum(jnp.where(sel0, rank_g, 0.0), axis=1)
    rank1 = jnp.sum(jnp.where(sel1, rank_g, 0.0), axis=1)
    e2_ref[...] = jnp.concatenate(
        [e0.reshape(1, tb), e1.reshape(1, tb)], axis=0)
    r2_ref[...] = jnp.concatenate(
        [rank0.reshape(1, tb), rank1.reshape(1, tb)], axis=0).astype(I32)


def _router(x, Wr):
    T, d = x.shape
    TB = 512
    nblk = T // TB
    body = functools.partial(_router_body, nblk=nblk, tb=TB)
    return pl.pallas_call(
        body,
        grid=(nblk,),
        in_specs=[
            pl.BlockSpec((TB, d), lambda g: (g, 0)),
            pl.BlockSpec((E, d), lambda g: (0, 0)),
        ],
        out_specs=[
            pl.BlockSpec((TB, E), lambda g: (g, 0)),
            pl.BlockSpec((TOP_K, TB), lambda g: (0, g)),
            pl.BlockSpec((TOP_K, TB), lambda g: (0, g)),
            pl.BlockSpec((TB, TOP_K), lambda g: (g, 0)),
            pl.BlockSpec((1, 16), lambda g: (0, 0)),
        ],
        out_shape=[
            jax.ShapeDtypeStruct((T, E), F32),
            jax.ShapeDtypeStruct((TOP_K, T), I32),
            jax.ShapeDtypeStruct((TOP_K, T), I32),
            jax.ShapeDtypeStruct((T, TOP_K), F32),
            jax.ShapeDtypeStruct((1, 16), I32),
        ],
        scratch_shapes=[pltpu.VMEM((1, 128), F32)],
        compiler_params=pltpu.CompilerParams(
            dimension_semantics=("arbitrary",)),
    )(x, Wr)


def _posmeta_body(cnt_ref, e2_ref, r2_ref, pos_ref, meta_ref, *, tb, n_meta):
    g = pl.program_id(0)
    offs = []
    ends = []
    off = 0
    for e in range(E):
        ce = cnt_ref[e]
        pce = ((ce + (B - 1)) >> LOG2B) << LOG2B
        offs.append(off)
        off = off + pce
        ends.append(off)

    eblk = e2_ref[...]
    acc = r2_ref[...]
    for e in range(E):
        acc = acc + jnp.where(eblk == e, offs[e], 0)
    pos_ref[...] = acc

    @pl.when(g == 0)
    def _():
        gv = lax.broadcasted_iota(I32, (1, n_meta), 1) * B
        accm = jnp.zeros((1, n_meta), I32)
        for e in range(E):
            accm = accm + (gv >= ends[e]).astype(I32)
        bev = jnp.minimum(accm, E - 1)
        nb = lax.shift_right_logical(ends[E - 1], LOG2B)
        lane = lax.broadcasted_iota(I32, (1, n_meta), 1)
        meta_ref[...] = jnp.where(lane == n_meta - 1, nb, bev)


def _posmeta(e2, r2, cnt16, n_meta):
    T = e2.shape[1]
    TB = 512
    grid_spec = pltpu.PrefetchScalarGridSpec(
        num_scalar_prefetch=1,
        grid=(T // TB,),
        in_specs=[
            pl.BlockSpec((TOP_K, TB), lambda g, c: (0, g)),
            pl.BlockSpec((TOP_K, TB), lambda g, c: (0, g)),
        ],
        out_specs=[
            pl.BlockSpec((TOP_K, TB), lambda g, c: (0, g)),
            pl.BlockSpec((1, n_meta), lambda g, c: (0, 0)),
        ],
    )
    return pl.pallas_call(
        functools.partial(_posmeta_body, tb=TB, n_meta=n_meta),
        grid_spec=grid_spec,
        out_shape=[
            jax.ShapeDtypeStruct((TOP_K, T), I32),
            jax.ShapeDtypeStruct((1, n_meta), I32),
        ],
        compiler_params=pltpu.CompilerParams(
            dimension_semantics=("arbitrary",)),
    )(cnt16, e2, r2)


def _dispatch(x, pos_flat, ns_tot):
    T, d = x.shape
    info = plsc.get_sparse_core_info()
    NC, NSUB = info.num_cores, info.num_subcores
    NW = NC * NSUB
    tok_w = T // NW
    n_ch = tok_w // 16
    mesh = plsc.VectorSubcoreMesh(core_axis_name="c", subcore_axis_name="s")

    @functools.partial(
        pl.kernel, mesh=mesh,
        out_type=jax.ShapeDtypeStruct((ns_tot, d), F32),
        scratch_types=[
            pltpu.VMEM((16,), I32),
            pltpu.VMEM((16,), I32),
            pltpu.VMEM((16, d), F32),
            pltpu.SemaphoreType.DMA,
            pltpu.SemaphoreType.DMA,
        ],
    )
    def disp(x_hbm, pos_hbm, xs_hbm, idx0, idx1, xv, sem0, sem1):
        wid = lax.axis_index("s") * NC + lax.axis_index("c")
        base = wid * tok_w
        for c in range(n_ch):
            pltpu.sync_copy(x_hbm.at[pl.ds(base + c * 16, 16)], xv)
            pltpu.sync_copy(pos_hbm.at[pl.ds(base + c * 16, 16)], idx0)
            pltpu.sync_copy(pos_hbm.at[pl.ds(T + base + c * 16, 16)], idx1)
            cp0 = pltpu.async_copy(xv, xs_hbm.at[idx0], sem0)
            cp1 = pltpu.async_copy(xv, xs_hbm.at[idx1], sem1)
            cp0.wait()
            cp1.wait()

    return disp(x, pos_flat)


def _k1_body(meta_ref, xs_ref, w1_ref, w3_ref, h_ref, *, n_meta):
    g = pl.program_id(1)

    @pl.when(g < meta_ref[n_meta - 1])
    def _():
        x = xs_ref[...]
        a = lax.dot_general(x, w1_ref[0], (((1,), (1,)), ((), ())),
                            preferred_element_type=F32)
        b = lax.dot_general(x, w3_ref[0], (((1,), (1,)), ((), ())),
                            preferred_element_type=F32)
        h_ref[...] = a * (1.0 / (1.0 + jnp.exp(-a))) * b


def _k1(meta, xs, W1, W3, n_meta):
    ns_tot, d = xs.shape
    f = W1.shape[1]
    FB = 2048
    n_fb = f // FB
    G = ns_tot // B
    grid_spec = pltpu.PrefetchScalarGridSpec(
        num_scalar_prefetch=1,
        grid=(n_fb, G),
        in_specs=[
            pl.BlockSpec(
                (B, d),
                lambda fb, g, m, n=n_meta: (jnp.minimum(g, m[n - 1] - 1), 0)),
            pl.BlockSpec((1, FB, d), lambda fb, g, m: (m[g], fb, 0)),
            pl.BlockSpec((1, FB, d), lambda fb, g, m: (m[g], fb, 0)),
        ],
        out_specs=pl.BlockSpec(
            (B, FB),
            lambda fb, g, m, n=n_meta: (jnp.minimum(g, m[n - 1] - 1), fb)),
    )
    return pl.pallas_call(
        functools.partial(_k1_body, n_meta=n_meta),
        grid_spec=grid_spec,
        out_shape=jax.ShapeDtypeStruct((ns_tot, f), F32),
        compiler_params=pltpu.CompilerParams(
            dimension_semantics=("arbitrary", "arbitrary")),
    )(meta, xs, W1, W3)


def _k2_body(meta_ref, h_ref, w2_ref, out_ref, *, n_meta):
    g = pl.program_id(0)

    @pl.when(g < meta_ref[n_meta - 1])
    def _():
        out_ref[...] = lax.dot_general(
            h_ref[...], w2_ref[0], (((1,), (1,)), ((), ())),
            preferred_element_type=F32)


def _k2(meta, h, W2, n_meta):
    ns_tot, f = h.shape
    d = W2.shape[1]
    G = ns_tot // B
    grid_spec = pltpu.PrefetchScalarGridSpec(
        num_scalar_prefetch=1,
        grid=(G,),
        in_specs=[
            pl.BlockSpec(
                (B, f),
                lambda g, m, n=n_meta: (jnp.minimum(g, m[n - 1] - 1), 0)),
            pl.BlockSpec((1, d, f), lambda g, m: (m[g], 0, 0)),
        ],
        out_specs=pl.BlockSpec(
            (B, d), lambda g, m, n=n_meta: (jnp.minimum(g, m[n - 1] - 1), 0)),
    )
    return pl.pallas_call(
        functools.partial(_k2_body, n_meta=n_meta),
        grid_spec=grid_spec,
        out_shape=jax.ShapeDtypeStruct((ns_tot, d), F32),
        compiler_params=pltpu.CompilerParams(
            dimension_semantics=("arbitrary",)),
    )(meta, h, W2)


def _undispatch(osort, pos_flat, T, d):
    info = plsc.get_sparse_core_info()
    NC, NSUB = info.num_cores, info.num_subcores
    NW = NC * NSUB
    tok_w = T // NW
    n_ch = tok_w // 16
    mesh = plsc.VectorSubcoreMesh(core_axis_name="c", subcore_axis_name="s")

    @functools.partial(
        pl.kernel, mesh=mesh,
        out_type=jax.ShapeDtypeStruct((TOP_K * T, d), F32),
        scratch_types=[
            pltpu.VMEM((16,), I32),
            pltpu.VMEM((16, d), F32),
            pltpu.SemaphoreType.DMA,
        ],
    )
    def undisp(os_hbm, pos_hbm, op_hbm, idx, buf, sem):
        wid = lax.axis_index("s") * NC + lax.axis_index("c")
        base = wid * tok_w
        for c in range(2 * n_ch):
            src = base + c * 16 if c < n_ch else T + base + (c - n_ch) * 16
            pltpu.sync_copy(pos_hbm.at[pl.ds(src, 16)], idx)
            pltpu.async_copy(os_hbm.at[idx], buf, sem).wait()
            pltpu.sync_copy(buf, op_hbm.at[pl.ds(src, 16)])

    return undisp(osort, pos_flat)


def _comb_body(op0_ref, op1_ref, w_ref, out_ref):
    w = w_ref[...]
    out_ref[...] = op0_ref[0] * w[:, 0:1] + op1_ref[0] * w[:, 1:2]


def _combine(op, wtok):
    _, T, d = op.shape
    TB = 512
    return pl.pallas_call(
        _comb_body,
        grid=(T // TB,),
        in_specs=[
            pl.BlockSpec((1, TB, d), lambda g: (0, g, 0)),
            pl.BlockSpec((1, TB, d), lambda g: (1, g, 0)),
            pl.BlockSpec((TB, TOP_K), lambda g: (g, 0)),
        ],
        out_specs=pl.BlockSpec((TB, d), lambda g: (g, 0)),
        out_shape=jax.ShapeDtypeStruct((T, d), F32),
        compiler_params=pltpu.CompilerParams(
            dimension_semantics=("arbitrary",)),
    )(op, op, wtok)


def kernel(hidden_states, Wr, W1, W3, W2):
    bs, S, d = hidden_states.shape
    T = bs * S
    ns_tot = TOP_K * T + E * B
    n_meta = 64
    x = hidden_states.reshape(-1, d)
    logits, e2, r2, wtok, counts = _router(x, Wr)
    pos, meta = _posmeta(e2, r2, counts.reshape(16), n_meta)
    pos_flat = pos.reshape(-1)
    meta_flat = meta.reshape(-1)
    xs = _dispatch(x, pos_flat, ns_tot)
    h = _k1(meta_flat, xs, W1, W3, n_meta)
    osort = _k2(meta_flat, h, W2, n_meta)
    op = _undispatch(osort, pos_flat, T, d)
    final = _combine(op.reshape(TOP_K, T, d), wtok)
    return final.reshape(bs, S, d), logits


# speedup vs baseline: 2.5823x; 1.0484x over previous
import functools

import jax
import jax.numpy as jnp
from jax import lax
from jax.experimental import pallas as pl
from jax.experimental.pallas import tpu as pltpu
from jax.experimental.pallas import tpu_sc as plsc

E = 8
TOP_K = 2
B = 512
F32 = jnp.float32
I32 = jnp.int32


def _router_body(x_ref, wr_ref, logits_ref, wtok_ref, pos_ref, meta_ref,
                 carry_ref, e_acc, r_acc, *, nblk, tb, n_meta, t_tot):
    g = pl.program_id(0)
    x = x_ref[...]
    logits = lax.dot_general(x, wr_ref[...], (((1,), (1,)), ((), ())),
                             preferred_element_type=F32)
    logits_ref[...] = logits

    m = jnp.max(logits, axis=1, keepdims=True)
    p = jnp.exp(logits - m)
    rw = p / jnp.sum(p, axis=1, keepdims=True)

    iota8 = lax.broadcasted_iota(I32, (tb, E), 1)
    m1 = jnp.max(rw, axis=1, keepdims=True)
    e0 = jnp.min(jnp.where(rw >= m1, iota8, E), axis=1, keepdims=True)
    sel0 = iota8 == e0
    rw2 = jnp.where(sel0, -jnp.inf, rw)
    m2 = jnp.max(rw2, axis=1, keepdims=True)
    e1 = jnp.min(jnp.where(rw2 >= m2, iota8, E), axis=1, keepdims=True)
    sel1 = iota8 == e1
    s = m1 + m2
    wtok_ref[...] = jnp.concatenate([m1 / s, m2 / s], axis=1)

    match = (sel0 | sel1).astype(F32)
    ri = lax.broadcasted_iota(I32, (tb, tb), 0)
    ci = lax.broadcasted_iota(I32, (tb, tb), 1)
    tri = (ri > ci).astype(F32)
    rank_in = lax.dot_general(tri, match, (((1,), (0,)), ((), ())),
                              preferred_element_type=F32)

    @pl.when(g == 0)
    def _():
        carry_ref[...] = jnp.zeros_like(carry_ref)

    carry = carry_ref[0:1, 0:E]
    rank_g = rank_in + carry
    colsum = jnp.sum(match, axis=0, keepdims=True)
    carry_ref[0:1, 0:E] = carry + colsum

    rank0 = jnp.sum(jnp.where(sel0, rank_g, 0.0), axis=1)
    rank1 = jnp.sum(jnp.where(sel1, rank_g, 0.0), axis=1)
    e_acc[:, pl.ds(g * tb, tb)] = jnp.concatenate(
        [e0.reshape(1, tb), e1.reshape(1, tb)], axis=0)
    r_acc[:, pl.ds(g * tb, tb)] = jnp.concatenate(
        [rank0.reshape(1, tb), rank1.reshape(1, tb)], axis=0).astype(I32)

    @pl.when(g == nblk - 1)
    def _():
        tot = carry + colsum
        pc = jnp.floor((tot + (B - 1)) * (1.0 / B)) * B
        rj = lax.broadcasted_iota(I32, (E, E), 0)
        cj = lax.broadcasted_iota(I32, (E, E), 1)
        triu = (rj <= cj).astype(F32)
        incl = lax.dot_general(pc, triu, (((1,), (0,)), ((), ())),
                               preferred_element_type=F32)
        offs = incl - pc
        lane8 = lax.broadcasted_iota(I32, (1, E), 1)

        eall = e_acc[...]
        acc = jnp.zeros(eall.shape, F32)
        for e in range(E):
            off_e = jnp.sum(offs * (lane8 == e), axis=1, keepdims=True)
            acc = acc + jnp.where(eall == e, 1.0, 0.0) * off_e
        pos_ref[...] = r_acc[...] + acc.astype(I32)

        gv = (lax.broadcasted_iota(I32, (1, n_meta), 1) * B).astype(F32)
        accm = jnp.zeros((1, n_meta), F32)
        for e in range(E):
            end_e = jnp.sum(incl * (lane8 == e), axis=1, keepdims=True)
            accm = accm + jnp.where(gv >= end_e, 1.0, 0.0)
        bev = jnp.minimum(accm, float(E - 1))
        nb = jnp.sum(incl * (lane8 == (E - 1)), axis=1, keepdims=True) * (1.0 / B)
        lane = lax.broadcasted_iota(I32, (1, n_meta), 1)
        meta_ref[...] = jnp.where(lane == n_meta - 1, nb, bev).astype(I32)


def _router(x, Wr, n_meta):
    T, d = x.shape
    TB = 512
    nblk = T // TB
    body = functools.partial(_router_body, nblk=nblk, tb=TB, n_meta=n_meta,
                             t_tot=T)
    return pl.pallas_call(
        body,
        grid=(nblk,),
        in_specs=[
            pl.BlockSpec((TB, d), lambda g: (g, 0)),
            pl.BlockSpec((E, d), lambda g: (0, 0)),
        ],
        out_specs=[
            pl.BlockSpec((TB, E), lambda g: (g, 0)),
            pl.BlockSpec((TB, TOP_K), lambda g: (g, 0)),
            pl.BlockSpec((TOP_K, T), lambda g: (0, 0)),
            pl.BlockSpec((1, n_meta), lambda g: (0, 0)),
        ],
        out_shape=[
            jax.ShapeDtypeStruct((T, E), F32),
            jax.ShapeDtypeStruct((T, TOP_K), F32),
            jax.ShapeDtypeStruct((TOP_K, T), I32),
            jax.ShapeDtypeStruct((1, n_meta), I32),
        ],
        scratch_shapes=[
            pltpu.VMEM((1, 128), F32),
            pltpu.VMEM((TOP_K, T), I32),
            pltpu.VMEM((TOP_K, T), I32),
        ],
        compiler_params=pltpu.CompilerParams(
            dimension_semantics=("arbitrary",)),
    )(x, Wr)


def _dispatch(x, pos_flat, ns_tot):
    T, d = x.shape
    info = plsc.get_sparse_core_info()
    NC, NSUB = info.num_cores, info.num_subcores
    NW = NC * NSUB
    tok_w = T // NW
    n_ch = tok_w // 16
    mesh = plsc.VectorSubcoreMesh(core_axis_name="c", subcore_axis_name="s")

    @functools.partial(
        pl.kernel, mesh=mesh,
        out_type=jax.ShapeDtypeStruct((ns_tot, d), F32),
        scratch_types=[
            pltpu.VMEM((16,), I32), pltpu.VMEM((16,), I32),
            pltpu.VMEM((16,), I32), pltpu.VMEM((16,), I32),
            pltpu.VMEM((16, d), F32), pltpu.VMEM((16, d), F32),
            pltpu.SemaphoreType.DMA, pltpu.SemaphoreType.DMA,
            pltpu.SemaphoreType.DMA, pltpu.SemaphoreType.DMA,
            pltpu.SemaphoreType.DMA, pltpu.SemaphoreType.DMA,
        ],
    )
    def disp(x_hbm, pos_hbm, xs_hbm, i0a, i0b, i1a, i1b, xva, xvb,
             sia, sib, s0a, s0b, s1a, s1b):
        wid = lax.axis_index("s") * NC + lax.axis_index("c")
        base = wid * tok_w
        idx0 = [i0a, i0b]
        idx1 = [i1a, i1b]
        xv = [xva, xvb]
        sin = [sia, sib]
        s0 = [s0a, s0b]
        s1 = [s1a, s1b]
        cp_in = [None, None]
        cp_s0 = [None, None]
        cp_s1 = [None, None]

        cp_in[0] = pltpu.async_copy(x_hbm.at[pl.ds(base, 16)], xv[0], sin[0])
        for c in range(n_ch):
            p = c % 2
            cp_in[p].wait()
            if c + 1 < n_ch:
                q = 1 - p
                if cp_s0[q] is not None:
                    cp_s0[q].wait()
                    cp_s1[q].wait()
                cp_in[q] = pltpu.async_copy(
                    x_hbm.at[pl.ds(base + (c + 1) * 16, 16)], xv[q], sin[q])
            pltpu.sync_copy(pos_hbm.at[pl.ds(base + c * 16, 16)], idx0[p])
            pltpu.sync_copy(pos_hbm.at[pl.ds(T + base + c * 16, 16)], idx1[p])
            cp_s0[p] = pltpu.async_copy(xv[p], xs_hbm.at[idx0[p]], s0[p])
            cp_s1[p] = pltpu.async_copy(xv[p], xs_hbm.at[idx1[p]], s1[p])
        for p in range(2):
            if cp_s0[p] is not None:
                cp_s0[p].wait()
                cp_s1[p].wait()

    return disp(x, pos_flat)


def _k1_body(meta_ref, xs_ref, w1_ref, w3_ref, h_ref, *, n_meta):
    g = pl.program_id(1)

    @pl.when(g < meta_ref[n_meta - 1])
    def _():
        x = xs_ref[...]
        a = lax.dot_general(x, w1_ref[0], (((1,), (1,)), ((), ())),
                            preferred_element_type=F32)
        b = lax.dot_general(x, w3_ref[0], (((1,), (1,)), ((), ())),
                            preferred_element_type=F32)
        h_ref[...] = a * (1.0 / (1.0 + jnp.exp(-a))) * b


def _k1(meta, xs, W1, W3, n_meta):
    ns_tot, d = xs.shape
    f = W1.shape[1]
    FB = 2048
    n_fb = f // FB
    G = ns_tot // B
    grid_spec = pltpu.PrefetchScalarGridSpec(
        num_scalar_prefetch=1,
        grid=(n_fb, G),
        in_specs=[
            pl.BlockSpec(
                (B, d),
                lambda fb, g, m, n=n_meta: (jnp.minimum(g, m[n - 1] - 1), 0)),
            pl.BlockSpec((1, FB, d), lambda fb, g, m: (m[g], fb, 0)),
            pl.BlockSpec((1, FB, d), lambda fb, g, m: (m[g], fb, 0)),
        ],
        out_specs=pl.BlockSpec(
            (B, FB),
            lambda fb, g, m, n=n_meta: (jnp.minimum(g, m[n - 1] - 1), fb)),
    )
    return pl.pallas_call(
        functools.partial(_k1_body, n_meta=n_meta),
        grid_spec=grid_spec,
        out_shape=jax.ShapeDtypeStruct((ns_tot, f), F32),
        compiler_params=pltpu.CompilerParams(
            dimension_semantics=("arbitrary", "arbitrary")),
    )(meta, xs, W1, W3)


def _k2_body(meta_ref, h_ref, w2_ref, out_ref, *, n_meta):
    g = pl.program_id(0)

    @pl.when(g < meta_ref[n_meta - 1])
    def _():
        out_ref[...] = lax.dot_general(
            h_ref[...], w2_ref[0], (((1,), (1,)), ((), ())),
            preferred_element_type=F32)


def _k2(meta, h, W2, n_meta):
    ns_tot, f = h.shape
    d = W2.shape[1]
    G = ns_tot // B
    grid_spec = pltpu.PrefetchScalarGridSpec(
        num_scalar_prefetch=1,
        grid=(G,),
        in_specs=[
            pl.BlockSpec(
                (B, f),
                lambda g, m, n=n_meta: (jnp.minimum(g, m[n - 1] - 1), 0)),
            pl.BlockSpec((1, d, f), lambda g, m: (m[g], 0, 0)),
        ],
        out_specs=pl.BlockSpec(
            (B, d), lambda g, m, n=n_meta: (jnp.minimum(g, m[n - 1] - 1), 0)),
    )
    return pl.pallas_call(
        functools.partial(_k2_body, n_meta=n_meta),
        grid_spec=grid_spec,
        out_shape=jax.ShapeDtypeStruct((ns_tot, d), F32),
        compiler_params=pltpu.CompilerParams(
            dimension_semantics=("arbitrary",)),
    )(meta, h, W2)


def _undispatch(osort, pos_flat, T, d):
    info = plsc.get_sparse_core_info()
    NC, NSUB = info.num_cores, info.num_subcores
    NW = NC * NSUB
    tok_w = T // NW
    n_ch = tok_w // 16
    mesh = plsc.VectorSubcoreMesh(core_axis_name="c", subcore_axis_name="s")

    @functools.partial(
        pl.kernel, mesh=mesh,
        out_type=jax.ShapeDtypeStruct((TOP_K * T, d), F32),
        scratch_types=[
            pltpu.VMEM((16,), I32), pltpu.VMEM((16,), I32),
            pltpu.VMEM((16, d), F32), pltpu.VMEM((16, d), F32),
            pltpu.SemaphoreType.DMA, pltpu.SemaphoreType.DMA,
        ],
    )
    def undisp(os_hbm, pos_hbm, op_hbm, ia, ib, bufa, bufb, sga, sgb):
        wid = lax.axis_index("s") * NC + lax.axis_index("c")
        base = wid * tok_w
        idx = [ia, ib]
        buf = [bufa, bufb]
        sg = [sga, sgb]
        cpg = [None, None]

        def src(c):
            return base + c * 16 if c < n_ch else T + base + (c - n_ch) * 16

        for c in range(2 * n_ch):
            p = c % 2
            if cpg[p] is not None:
                cpg[p].wait()
                pltpu.sync_copy(buf[p], op_hbm.at[pl.ds(src(c - 2), 16)])
            pltpu.sync_copy(pos_hbm.at[pl.ds(src(c), 16)], idx[p])
            cpg[p] = pltpu.async_copy(os_hbm.at[idx[p]], buf[p], sg[p])
        for p in range(2):
            c_last = 2 * n_ch - 2 + p
            cpg[p].wait()
            pltpu.sync_copy(buf[p], op_hbm.at[pl.ds(src(c_last), 16)])

    return undisp(osort, pos_flat)


def _comb_body(op0_ref, op1_ref, w_ref, out_ref):
    w = w_ref[...]
    out_ref[...] = op0_ref[0] * w[:, 0:1] + op1_ref[0] * w[:, 1:2]


def _combine(op, wtok):
    _, T, d = op.shape
    TB = 512
    return pl.pallas_call(
        _comb_body,
        grid=(T // TB,),
        in_specs=[
            pl.BlockSpec((1, TB, d), lambda g: (0, g, 0)),
            pl.BlockSpec((1, TB, d), lambda g: (1, g, 0)),
            pl.BlockSpec((TB, TOP_K), lambda g: (g, 0)),
        ],
        out_specs=pl.BlockSpec((TB, d), lambda g: (g, 0)),
        out_shape=jax.ShapeDtypeStruct((T, d), F32),
        compiler_params=pltpu.CompilerParams(
            dimension_semantics=("arbitrary",)),
    )(op, op, wtok)


def kernel(hidden_states, Wr, W1, W3, W2):
    bs, S, d = hidden_states.shape
    T = bs * S
    ns_tot = TOP_K * T + E * B
    n_meta = ns_tot // B + 8
    x = hidden_states.reshape(-1, d)
    logits, wtok, pos, meta = _router(x, Wr, n_meta)
    pos_flat = pos.reshape(-1)
    meta_flat = meta.reshape(-1)
    xs = _dispatch(x, pos_flat, ns_tot)
    h = _k1(meta_flat, xs, W1, W3, n_meta)
    osort = _k2(meta_flat, h, W2, n_meta)
    op = _undispatch(osort, pos_flat, T, d)
    final = _combine(op.reshape(TOP_K, T, d), wtok)
    return final.reshape(bs, S, d), logits


# speedup vs baseline: 2.6429x; 1.0234x over previous
import functools

import jax
import jax.numpy as jnp
from jax import lax
from jax.experimental import pallas as pl
from jax.experimental.pallas import tpu as pltpu
from jax.experimental.pallas import tpu_sc as plsc

E = 8
TOP_K = 2
B = 512
F32 = jnp.float32
I32 = jnp.int32
BF16 = jnp.bfloat16


def _router_body(x_ref, wr_ref, logits_ref, wtok_ref, pos_ref, meta_ref,
                 carry_ref, e_acc, r_acc, *, nblk, tb, n_meta, t_tot):
    g = pl.program_id(0)
    x = x_ref[...]
    logits = lax.dot_general(x, wr_ref[...], (((1,), (1,)), ((), ())),
                             preferred_element_type=F32)
    logits_ref[...] = logits

    m = jnp.max(logits, axis=1, keepdims=True)
    p = jnp.exp(logits - m)
    rw = p / jnp.sum(p, axis=1, keepdims=True)

    iota8 = lax.broadcasted_iota(I32, (tb, E), 1)
    m1 = jnp.max(rw, axis=1, keepdims=True)
    e0 = jnp.min(jnp.where(rw >= m1, iota8, E), axis=1, keepdims=True)
    sel0 = iota8 == e0
    rw2 = jnp.where(sel0, -jnp.inf, rw)
    m2 = jnp.max(rw2, axis=1, keepdims=True)
    e1 = jnp.min(jnp.where(rw2 >= m2, iota8, E), axis=1, keepdims=True)
    sel1 = iota8 == e1
    s = m1 + m2
    wtok_ref[...] = jnp.concatenate([m1 / s, m2 / s], axis=1)

    match = (sel0 | sel1).astype(F32)
    ri = lax.broadcasted_iota(I32, (tb, tb), 0)
    ci = lax.broadcasted_iota(I32, (tb, tb), 1)
    tri = (ri > ci).astype(F32)
    rank_in = lax.dot_general(tri, match, (((1,), (0,)), ((), ())),
                              preferred_element_type=F32)

    @pl.when(g == 0)
    def _():
        carry_ref[...] = jnp.zeros_like(carry_ref)

    carry = carry_ref[0:1, 0:E]
    rank_g = rank_in + carry
    colsum = jnp.sum(match, axis=0, keepdims=True)
    carry_ref[0:1, 0:E] = carry + colsum

    rank0 = jnp.sum(jnp.where(sel0, rank_g, 0.0), axis=1)
    rank1 = jnp.sum(jnp.where(sel1, rank_g, 0.0), axis=1)
    e_acc[:, pl.ds(g * tb, tb)] = jnp.concatenate(
        [e0.reshape(1, tb), e1.reshape(1, tb)], axis=0)
    r_acc[:, pl.ds(g * tb, tb)] = jnp.concatenate(
        [rank0.reshape(1, tb), rank1.reshape(1, tb)], axis=0).astype(I32)

    @pl.when(g == nblk - 1)
    def _():
        tot = carry + colsum
        pc = jnp.floor((tot + (B - 1)) * (1.0 / B)) * B
        rj = lax.broadcasted_iota(I32, (E, E), 0)
        cj = lax.broadcasted_iota(I32, (E, E), 1)
        triu = (rj <= cj).astype(F32)
        incl = lax.dot_general(pc, triu, (((1,), (0,)), ((), ())),
                               preferred_element_type=F32)
        offs = incl - pc
        lane8 = lax.broadcasted_iota(I32, (1, E), 1)

        eall = e_acc[...]
        acc = jnp.zeros(eall.shape, F32)
        for e in range(E):
            off_e = jnp.sum(offs * (lane8 == e), axis=1, keepdims=True)
            acc = acc + jnp.where(eall == e, 1.0, 0.0) * off_e
        pos_ref[...] = r_acc[...] + acc.astype(I32)

        gv = (lax.broadcasted_iota(I32, (1, n_meta), 1) * B).astype(F32)
        accm = jnp.zeros((1, n_meta), F32)
        for e in range(E):
            end_e = jnp.sum(incl * (lane8 == e), axis=1, keepdims=True)
            accm = accm + jnp.where(gv >= end_e, 1.0, 0.0)
        bev = jnp.minimum(accm, float(E - 1))
        nb = jnp.sum(incl * (lane8 == (E - 1)), axis=1, keepdims=True) * (1.0 / B)
        lane = lax.broadcasted_iota(I32, (1, n_meta), 1)
        meta_ref[...] = jnp.where(lane == n_meta - 1, nb, bev).astype(I32)


def _router(x, Wr, n_meta):
    T, d = x.shape
    TB = 512
    nblk = T // TB
    body = functools.partial(_router_body, nblk=nblk, tb=TB, n_meta=n_meta,
                             t_tot=T)
    return pl.pallas_call(
        body,
        grid=(nblk,),
        in_specs=[
            pl.BlockSpec((TB, d), lambda g: (g, 0)),
            pl.BlockSpec((E, d), lambda g: (0, 0)),
        ],
        out_specs=[
            pl.BlockSpec((TB, E), lambda g: (g, 0)),
            pl.BlockSpec((TB, TOP_K), lambda g: (g, 0)),
            pl.BlockSpec((TOP_K, T), lambda g: (0, 0)),
            pl.BlockSpec((1, n_meta), lambda g: (0, 0)),
        ],
        out_shape=[
            jax.ShapeDtypeStruct((T, E), F32),
            jax.ShapeDtypeStruct((T, TOP_K), F32),
            jax.ShapeDtypeStruct((TOP_K, T), I32),
            jax.ShapeDtypeStruct((1, n_meta), I32),
        ],
        scratch_shapes=[
            pltpu.VMEM((1, 128), F32),
            pltpu.VMEM((TOP_K, T), I32),
            pltpu.VMEM((TOP_K, T), I32),
        ],
        compiler_params=pltpu.CompilerParams(
            dimension_semantics=("arbitrary",)),
    )(x, Wr)


def _dispatch(x, pos_flat, ns_tot):
    T, d = x.shape
    info = plsc.get_sparse_core_info()
    NC, NSUB = info.num_cores, info.num_subcores
    NW = NC * NSUB
    tok_w = T // NW
    n_ch = tok_w // 16
    mesh = plsc.VectorSubcoreMesh(core_axis_name="c", subcore_axis_name="s")

    @functools.partial(
        pl.kernel, mesh=mesh,
        out_type=jax.ShapeDtypeStruct((ns_tot, d), F32),
        scratch_types=[
            pltpu.VMEM((16,), I32), pltpu.VMEM((16,), I32),
            pltpu.VMEM((16,), I32), pltpu.VMEM((16,), I32),
            pltpu.VMEM((16, d), F32), pltpu.VMEM((16, d), F32),
            pltpu.SemaphoreType.DMA, pltpu.SemaphoreType.DMA,
            pltpu.SemaphoreType.DMA, pltpu.SemaphoreType.DMA,
            pltpu.SemaphoreType.DMA, pltpu.SemaphoreType.DMA,
        ],
    )
    def disp(x_hbm, pos_hbm, xs_hbm, i0a, i0b, i1a, i1b, xva, xvb,
             sia, sib, s0a, s0b, s1a, s1b):
        wid = lax.axis_index("s") * NC + lax.axis_index("c")
        base = wid * tok_w
        idx0 = [i0a, i0b]
        idx1 = [i1a, i1b]
        xv = [xva, xvb]
        sin = [sia, sib]
        s0 = [s0a, s0b]
        s1 = [s1a, s1b]
        cp_in = [None, None]
        cp_s0 = [None, None]
        cp_s1 = [None, None]

        cp_in[0] = pltpu.async_copy(x_hbm.at[pl.ds(base, 16)], xv[0], sin[0])
        for c in range(n_ch):
            p = c % 2
            cp_in[p].wait()
            if c + 1 < n_ch:
                q = 1 - p
                if cp_s0[q] is not None:
                    cp_s0[q].wait()
                    cp_s1[q].wait()
                cp_in[q] = pltpu.async_copy(
                    x_hbm.at[pl.ds(base + (c + 1) * 16, 16)], xv[q], sin[q])
            pltpu.sync_copy(pos_hbm.at[pl.ds(base + c * 16, 16)], idx0[p])
            pltpu.sync_copy(pos_hbm.at[pl.ds(T + base + c * 16, 16)], idx1[p])
            cp_s0[p] = pltpu.async_copy(xv[p], xs_hbm.at[idx0[p]], s0[p])
            cp_s1[p] = pltpu.async_copy(xv[p], xs_hbm.at[idx1[p]], s1[p])
        for p in range(2):
            if cp_s0[p] is not None:
                cp_s0[p].wait()
                cp_s1[p].wait()

    return disp(x, pos_flat)


def _k1_body(meta_ref, xs_ref, w1_ref, w3_ref, h_ref, w1b_ref, w3b_ref,
             *, n_meta):
    g = pl.program_id(1)

    @pl.when(g < meta_ref[n_meta - 1])
    def _():
        gm1 = jnp.maximum(g - 1, 0)
        changed = (g == 0) | (meta_ref[g] != meta_ref[gm1])

        @pl.when(changed)
        def _():
            w1b_ref[...] = w1_ref[0].astype(BF16)
            w3b_ref[...] = w3_ref[0].astype(BF16)

        xb = xs_ref[...].astype(BF16)
        a = lax.dot_general(xb, w1b_ref[...], (((1,), (1,)), ((), ())),
                            preferred_element_type=F32)
        b = lax.dot_general(xb, w3b_ref[...], (((1,), (1,)), ((), ())),
                            preferred_element_type=F32)
        h_ref[...] = (a * (1.0 / (1.0 + jnp.exp(-a))) * b).astype(BF16)


def _k1(meta, xs, W1, W3, n_meta):
    ns_tot, d = xs.shape
    f = W1.shape[1]
    FB = 2048
    n_fb = f // FB
    G = ns_tot // B
    grid_spec = pltpu.PrefetchScalarGridSpec(
        num_scalar_prefetch=1,
        grid=(n_fb, G),
        in_specs=[
            pl.BlockSpec(
                (B, d),
                lambda fb, g, m, n=n_meta: (jnp.minimum(g, m[n - 1] - 1), 0)),
            pl.BlockSpec((1, FB, d), lambda fb, g, m: (m[g], fb, 0)),
            pl.BlockSpec((1, FB, d), lambda fb, g, m: (m[g], fb, 0)),
        ],
        out_specs=pl.BlockSpec(
            (B, FB),
            lambda fb, g, m, n=n_meta: (jnp.minimum(g, m[n - 1] - 1), fb)),
        scratch_shapes=[
            pltpu.VMEM((FB, d), BF16),
            pltpu.VMEM((FB, d), BF16),
        ],
    )
    return pl.pallas_call(
        functools.partial(_k1_body, n_meta=n_meta),
        grid_spec=grid_spec,
        out_shape=jax.ShapeDtypeStruct((ns_tot, f), BF16),
        compiler_params=pltpu.CompilerParams(
            dimension_semantics=("arbitrary", "arbitrary")),
    )(meta, xs, W1, W3)


def _k2_body(meta_ref, h_ref, w2_ref, out_ref, w2b_ref, *, n_meta):
    g = pl.program_id(0)

    @pl.when(g < meta_ref[n_meta - 1])
    def _():
        gm1 = jnp.maximum(g - 1, 0)
        changed = (g == 0) | (meta_ref[g] != meta_ref[gm1])

        @pl.when(changed)
        def _():
            w2b_ref[...] = w2_ref[0].astype(BF16)

        out_ref[...] = lax.dot_general(
            h_ref[...], w2b_ref[...], (((1,), (1,)), ((), ())),
            preferred_element_type=F32)


def _k2(meta, h, W2, n_meta):
    ns_tot, f = h.shape
    d = W2.shape[1]
    G = ns_tot // B
    grid_spec = pltpu.PrefetchScalarGridSpec(
        num_scalar_prefetch=1,
        grid=(G,),
        in_specs=[
            pl.BlockSpec(
                (B, f),
                lambda g, m, n=n_meta: (jnp.minimum(g, m[n - 1] - 1), 0)),
            pl.BlockSpec((1, d, f), lambda g, m: (m[g], 0, 0)),
        ],
        out_specs=pl.BlockSpec(
            (B, d), lambda g, m, n=n_meta: (jnp.minimum(g, m[n - 1] - 1), 0)),
        scratch_shapes=[pltpu.VMEM((d, f), BF16)],
    )
    return pl.pallas_call(
        functools.partial(_k2_body, n_meta=n_meta),
        grid_spec=grid_spec,
        out_shape=jax.ShapeDtypeStruct((ns_tot, d), F32),
        compiler_params=pltpu.CompilerParams(
            dimension_semantics=("arbitrary",)),
    )(meta, h, W2)


def _undispatch(osort, pos_flat, T, d):
    info = plsc.get_sparse_core_info()
    NC, NSUB = info.num_cores, info.num_subcores
    NW = NC * NSUB
    tok_w = T // NW
    n_ch = tok_w // 16
    mesh = plsc.VectorSubcoreMesh(core_axis_name="c", subcore_axis_name="s")

    @functools.partial(
        pl.kernel, mesh=mesh,
        out_type=jax.ShapeDtypeStruct((TOP_K * T, d), F32),
        scratch_types=[
            pltpu.VMEM((16,), I32), pltpu.VMEM((16,), I32),
            pltpu.VMEM((16, d), F32), pltpu.VMEM((16, d), F32),
            pltpu.SemaphoreType.DMA, pltpu.SemaphoreType.DMA,
        ],
    )
    def undisp(os_hbm, pos_hbm, op_hbm, ia, ib, bufa, bufb, sga, sgb):
        wid = lax.axis_index("s") * NC + lax.axis_index("c")
        base = wid * tok_w
        idx = [ia, ib]
        buf = [bufa, bufb]
        sg = [sga, sgb]
        cpg = [None, None]

        def src(c):
            return base + c * 16 if c < n_ch else T + base + (c - n_ch) * 16

        for c in range(2 * n_ch):
            p = c % 2
            if cpg[p] is not None:
                cpg[p].wait()
                pltpu.sync_copy(buf[p], op_hbm.at[pl.ds(src(c - 2), 16)])
            pltpu.sync_copy(pos_hbm.at[pl.ds(src(c), 16)], idx[p])
            cpg[p] = pltpu.async_copy(os_hbm.at[idx[p]], buf[p], sg[p])
        for p in range(2):
            c_last = 2 * n_ch - 2 + p
            cpg[p].wait()
            pltpu.sync_copy(buf[p], op_hbm.at[pl.ds(src(c_last), 16)])

    return undisp(osort, pos_flat)


def _comb_body(op0_ref, op1_ref, w_ref, out_ref):
    w = w_ref[...]
    out_ref[...] = op0_ref[0] * w[:, 0:1] + op1_ref[0] * w[:, 1:2]


def _combine(op, wtok):
    _, T, d = op.shape
    TB = 512
    return pl.pallas_call(
        _comb_body,
        grid=(T // TB,),
        in_specs=[
            pl.BlockSpec((1, TB, d), lambda g: (0, g, 0)),
            pl.BlockSpec((1, TB, d), lambda g: (1, g, 0)),
            pl.BlockSpec((TB, TOP_K), lambda g: (g, 0)),
        ],
        out_specs=pl.BlockSpec((TB, d), lambda g: (g, 0)),
        out_shape=jax.ShapeDtypeStruct((T, d), F32),
        compiler_params=pltpu.CompilerParams(
            dimension_semantics=("arbitrary",)),
    )(op, op, wtok)


def kernel(hidden_states, Wr, W1, W3, W2):
    bs, S, d = hidden_states.shape
    T = bs * S
    ns_tot = TOP_K * T + E * B
    n_meta = ns_tot // B + 8
    x = hidden_states.reshape(-1, d)
    logits, wtok, pos, meta = _router(x, Wr, n_meta)
    pos_flat = pos.reshape(-1)
    meta_flat = meta.reshape(-1)
    xs = _dispatch(x, pos_flat, ns_tot)
    h = _k1(meta_flat, xs, W1, W3, n_meta)
    osort = _k2(meta_flat, h, W2, n_meta)
    op = _undispatch(osort, pos_flat, T, d)
    final = _combine(op.reshape(TOP_K, T, d), wtok)
    return final.reshape(bs, S, d), logits
